# baseline (speedup 1.0000x reference)
import jax
import jax.numpy as jnp
from jax.experimental import pallas as pl
from jax.experimental.pallas import tpu as pltpu

_TN = 1024
_CHUNK = 32


def _round_up(x, m):
    return ((x + m - 1) // m) * m


def _gather_dot_kernel(pids_ref, uids_ref, ptab_ref, utab_ref, out_ref, c_ref):
    tn = out_ref.shape[1]

    def chunk_body(c, carry):
        base = c * _CHUNK
        blks = []
        for j in range(_CHUNK // 8):
            rows = []
            for i in range(8):
                p = pids_ref[0, base + j * 8 + i]
                u = uids_ref[0, base + j * 8 + i]
                rows.append(ptab_ref[p] * utab_ref[u])
            blks.append(jnp.concatenate(rows, axis=0))
        blk = jnp.concatenate(blks, axis=0)
        c_ref[pl.ds(pl.multiple_of(base, _CHUNK), _CHUNK), :] = (
            blk.astype(jnp.bfloat16))
        return carry

    half = tn // (2 * _CHUNK)
    ones = jnp.ones((1, 128), jnp.bfloat16)
    dims = (((1,), (1,)), ((), ()))

    jax.lax.fori_loop(0, half, chunk_body, 0)
    out_ref[0, pl.ds(0, tn // 2)] = jax.lax.dot_general(
        ones, c_ref[pl.ds(0, tn // 2), :], dims,
        preferred_element_type=jnp.float32)[0]
    jax.lax.fori_loop(half, 2 * half, chunk_body, 0)
    out_ref[0, pl.ds(tn // 2, tn // 2)] = jax.lax.dot_general(
        ones, c_ref[pl.ds(tn // 2, tn // 2), :], dims,
        preferred_element_type=jnp.float32)[0]


def kernel(products, users, product_embedding, user_embedding):
    n = products.shape[0]
    p_rows, d = product_embedding.shape
    u_rows, d_u = user_embedding.shape
    assert d == d_u == 128

    n_pad = _round_up(n, _TN)
    num_tiles = n_pad // _TN

    def prep_ids(ids, rows):
        ids = jnp.clip(jnp.asarray(ids).astype(jnp.int32), 0, rows - 1)
        ids = jnp.pad(ids, (0, n_pad - n))
        return ids.reshape(num_tiles, 1, _TN)

    prod_ids = prep_ids(products, p_rows)
    user_ids = prep_ids(users, u_rows)

    ptab = product_embedding.astype(jnp.float32).reshape(p_rows, 1, d)
    utab = user_embedding.astype(jnp.float32).reshape(u_rows, 1, d)

    table_bytes = (p_rows + u_rows) * d * 4
    vmem_limit = min(int(2 * table_bytes + 4 * _TN * 128 * 4 + (8 << 20)),
                     60 << 20)

    cost = pl.CostEstimate(
        flops=2 * n_pad * d + 2 * n_pad * d,
        transcendentals=0,
        bytes_accessed=2 * n_pad * 4 + 2 * table_bytes + n_pad * 4,
    )

    out = pl.pallas_call(
        _gather_dot_kernel,
        out_shape=jax.ShapeDtypeStruct((num_tiles, 1, _TN), jnp.float32),
        grid=(num_tiles,),
        in_specs=[
            pl.BlockSpec((None, 1, _TN), lambda t: (t, 0, 0),
                         memory_space=pltpu.SMEM),
            pl.BlockSpec((None, 1, _TN), lambda t: (t, 0, 0),
                         memory_space=pltpu.SMEM),
            pl.BlockSpec((p_rows, 1, d), lambda t: (0, 0, 0)),
            pl.BlockSpec((u_rows, 1, d), lambda t: (0, 0, 0)),
        ],
        out_specs=pl.BlockSpec((None, 1, _TN), lambda t: (t, 0, 0)),
        scratch_shapes=[pltpu.VMEM((_TN, d), jnp.bfloat16)],
        compiler_params=pltpu.CompilerParams(
            dimension_semantics=("parallel",),
            vmem_limit_bytes=vmem_limit,
        ),
        cost_estimate=cost,
    )(prod_ids, user_ids, ptab, utab)
    return out.reshape(n_pad)[:n]


# speedup vs baseline: 1.1113x; 1.1113x over previous
import jax
import jax.numpy as jnp
from jax.experimental import pallas as pl
from jax.experimental.pallas import tpu as pltpu

_TN = 1024
_CHUNK = 32


def _round_up(x, m):
    return ((x + m - 1) // m) * m


def _gather_dot_kernel(pids_ref, uids_ref, ptab_ref, utab_ref, out_ref, c_ref):
    tn = out_ref.shape[1]

    def chunk_body(c, carry):
        base = c * _CHUNK
        for j in range(_CHUNK // 8):
            rows = []
            for i in range(8):
                p = pids_ref[0, base + j * 8 + i]
                u = uids_ref[0, base + j * 8 + i]
                rows.append(ptab_ref[p] * utab_ref[u])
            blk = jnp.concatenate(rows, axis=0)
            c_ref[pl.ds(pl.multiple_of(base + j * 8, 8), 8), :] = blk
        return carry

    half = tn // (2 * _CHUNK)
    ones = jnp.ones((1, 128), jnp.float32)
    dims = (((1,), (1,)), ((), ()))

    jax.lax.fori_loop(0, half, chunk_body, 0)
    out_ref[0, pl.ds(0, tn // 2)] = jax.lax.dot_general(
        ones, c_ref[pl.ds(0, tn // 2), :], dims,
        preferred_element_type=jnp.float32)[0]
    jax.lax.fori_loop(half, 2 * half, chunk_body, 0)
    out_ref[0, pl.ds(tn // 2, tn // 2)] = jax.lax.dot_general(
        ones, c_ref[pl.ds(tn // 2, tn // 2), :], dims,
        preferred_element_type=jnp.float32)[0]


def kernel(products, users, product_embedding, user_embedding):
    n = products.shape[0]
    p_rows, d = product_embedding.shape
    u_rows, d_u = user_embedding.shape
    assert d == d_u == 128

    n_pad = _round_up(n, _TN)
    num_tiles = n_pad // _TN

    def prep_ids(ids, rows):
        ids = jnp.clip(jnp.asarray(ids).astype(jnp.int32), 0, rows - 1)
        ids = jnp.pad(ids, (0, n_pad - n))
        return ids.reshape(num_tiles, 1, _TN)

    prod_ids = prep_ids(products, p_rows)
    user_ids = prep_ids(users, u_rows)

    ptab = product_embedding.astype(jnp.float32).reshape(p_rows, 1, d)
    utab = user_embedding.astype(jnp.float32).reshape(u_rows, 1, d)

    table_bytes = (p_rows + u_rows) * d * 4
    vmem_limit = min(int(2 * table_bytes + 4 * _TN * 128 * 4 + (8 << 20)),
                     60 << 20)

    cost = pl.CostEstimate(
        flops=2 * n_pad * d + 2 * n_pad * d,
        transcendentals=0,
        bytes_accessed=2 * n_pad * 4 + 2 * table_bytes + n_pad * 4,
    )

    out = pl.pallas_call(
        _gather_dot_kernel,
        out_shape=jax.ShapeDtypeStruct((num_tiles, 1, _TN), jnp.float32),
        grid=(num_tiles,),
        in_specs=[
            pl.BlockSpec((None, 1, _TN), lambda t: (t, 0, 0),
                         memory_space=pltpu.SMEM),
            pl.BlockSpec((None, 1, _TN), lambda t: (t, 0, 0),
                         memory_space=pltpu.SMEM),
            pl.BlockSpec((p_rows, 1, d), lambda t: (0, 0, 0)),
            pl.BlockSpec((u_rows, 1, d), lambda t: (0, 0, 0)),
        ],
        out_specs=pl.BlockSpec((None, 1, _TN), lambda t: (t, 0, 0)),
        scratch_shapes=[pltpu.VMEM((_TN, d), jnp.float32)],
        compiler_params=pltpu.CompilerParams(
            dimension_semantics=("parallel",),
            vmem_limit_bytes=vmem_limit,
        ),
        cost_estimate=cost,
    )(prod_ids, user_ids, ptab, utab)
    return out.reshape(n_pad)[:n]


# speedup vs baseline: 1.1735x; 1.0560x over previous
import jax
import jax.numpy as jnp
from jax.experimental import pallas as pl
from jax.experimental.pallas import tpu as pltpu

_TN = 1024
_CHUNK = 64


def _round_up(x, m):
    return ((x + m - 1) // m) * m


def _gather_dot_kernel(pids_ref, uids_ref, ptab_ref, utab_ref, out_ref, c_ref):
    tn = out_ref.shape[1]

    def chunk_body(c, carry):
        base = c * _CHUNK
        for j in range(_CHUNK // 8):
            rows = []
            for i in range(8):
                p = pids_ref[0, base + j * 8 + i]
                u = uids_ref[0, base + j * 8 + i]
                rows.append(ptab_ref[p] * utab_ref[u])
            blk = jnp.concatenate(rows, axis=0)
            c_ref[pl.ds(pl.multiple_of(base + j * 8, 8), 8), :] = blk
        return carry

    half = tn // (2 * _CHUNK)
    ones = jnp.ones((1, 128), jnp.float32)
    dims = (((1,), (1,)), ((), ()))

    jax.lax.fori_loop(0, half, chunk_body, 0)
    out_ref[0, pl.ds(0, tn // 2)] = jax.lax.dot_general(
        ones, c_ref[pl.ds(0, tn // 2), :], dims,
        preferred_element_type=jnp.float32)[0]
    jax.lax.fori_loop(half, 2 * half, chunk_body, 0)
    out_ref[0, pl.ds(tn // 2, tn // 2)] = jax.lax.dot_general(
        ones, c_ref[pl.ds(tn // 2, tn // 2), :], dims,
        preferred_element_type=jnp.float32)[0]


def kernel(products, users, product_embedding, user_embedding):
    n = products.shape[0]
    p_rows, d = product_embedding.shape
    u_rows, d_u = user_embedding.shape
    assert d == d_u == 128

    n_pad = _round_up(n, _TN)
    num_tiles = n_pad // _TN

    def prep_ids(ids, rows):
        ids = jnp.clip(jnp.asarray(ids).astype(jnp.int32), 0, rows - 1)
        ids = jnp.pad(ids, (0, n_pad - n))
        return ids.reshape(num_tiles, 1, _TN)

    prod_ids = prep_ids(products, p_rows)
    user_ids = prep_ids(users, u_rows)

    ptab = product_embedding.astype(jnp.float32).reshape(p_rows, 1, d)
    utab = user_embedding.astype(jnp.float32).reshape(u_rows, 1, d)

    table_bytes = (p_rows + u_rows) * d * 4
    vmem_limit = min(int(2 * table_bytes + 4 * _TN * 128 * 4 + (8 << 20)),
                     60 << 20)

    cost = pl.CostEstimate(
        flops=2 * n_pad * d + 2 * n_pad * d,
        transcendentals=0,
        bytes_accessed=2 * n_pad * 4 + 2 * table_bytes + n_pad * 4,
    )

    out = pl.pallas_call(
        _gather_dot_kernel,
        out_shape=jax.ShapeDtypeStruct((num_tiles, 1, _TN), jnp.float32),
        grid=(num_tiles,),
        in_specs=[
            pl.BlockSpec((None, 1, _TN), lambda t: (t, 0, 0),
                         memory_space=pltpu.SMEM),
            pl.BlockSpec((None, 1, _TN), lambda t: (t, 0, 0),
                         memory_space=pltpu.SMEM),
            pl.BlockSpec((p_rows, 1, d), lambda t: (0, 0, 0)),
            pl.BlockSpec((u_rows, 1, d), lambda t: (0, 0, 0)),
        ],
        out_specs=pl.BlockSpec((None, 1, _TN), lambda t: (t, 0, 0)),
        scratch_shapes=[pltpu.VMEM((_TN, d), jnp.float32)],
        compiler_params=pltpu.CompilerParams(
            dimension_semantics=("parallel",),
            vmem_limit_bytes=vmem_limit,
        ),
        cost_estimate=cost,
    )(prod_ids, user_ids, ptab, utab)
    return out.reshape(n_pad)[:n]


# speedup vs baseline: 1.2097x; 1.0309x over previous
import jax
import jax.numpy as jnp
from jax.experimental import pallas as pl
from jax.experimental.pallas import tpu as pltpu

_TN = 1024
_CHUNK = 128


def _round_up(x, m):
    return ((x + m - 1) // m) * m


def _gather_dot_kernel(pids_ref, uids_ref, ptab_ref, utab_ref, out_ref, c_ref):
    tn = out_ref.shape[1]

    def chunk_body(c, carry):
        base = c * _CHUNK
        for j in range(_CHUNK // 8):
            rows = []
            for i in range(8):
                p = pids_ref[0, base + j * 8 + i]
                u = uids_ref[0, base + j * 8 + i]
                rows.append(ptab_ref[p] * utab_ref[u])
            blk = jnp.concatenate(rows, axis=0)
            c_ref[pl.ds(pl.multiple_of(base + j * 8, 8), 8), :] = blk
        return carry

    half = tn // (2 * _CHUNK)
    ones = jnp.ones((1, 128), jnp.float32)
    dims = (((1,), (1,)), ((), ()))

    jax.lax.fori_loop(0, half, chunk_body, 0)
    out_ref[0, pl.ds(0, tn // 2)] = jax.lax.dot_general(
        ones, c_ref[pl.ds(0, tn // 2), :], dims,
        preferred_element_type=jnp.float32)[0]
    jax.lax.fori_loop(half, 2 * half, chunk_body, 0)
    out_ref[0, pl.ds(tn // 2, tn // 2)] = jax.lax.dot_general(
        ones, c_ref[pl.ds(tn // 2, tn // 2), :], dims,
        preferred_element_type=jnp.float32)[0]


def kernel(products, users, product_embedding, user_embedding):
    n = products.shape[0]
    p_rows, d = product_embedding.shape
    u_rows, d_u = user_embedding.shape
    assert d == d_u == 128

    n_pad = _round_up(n, _TN)
    num_tiles = n_pad // _TN

    def prep_ids(ids, rows):
        ids = jnp.clip(jnp.asarray(ids).astype(jnp.int32), 0, rows - 1)
        ids = jnp.pad(ids, (0, n_pad - n))
        return ids.reshape(num_tiles, 1, _TN)

    prod_ids = prep_ids(products, p_rows)
    user_ids = prep_ids(users, u_rows)

    ptab = product_embedding.astype(jnp.float32).reshape(p_rows, 1, d)
    utab = user_embedding.astype(jnp.float32).reshape(u_rows, 1, d)

    table_bytes = (p_rows + u_rows) * d * 4
    vmem_limit = min(int(2 * table_bytes + 4 * _TN * 128 * 4 + (8 << 20)),
                     60 << 20)

    cost = pl.CostEstimate(
        flops=2 * n_pad * d + 2 * n_pad * d,
        transcendentals=0,
        bytes_accessed=2 * n_pad * 4 + 2 * table_bytes + n_pad * 4,
    )

    out = pl.pallas_call(
        _gather_dot_kernel,
        out_shape=jax.ShapeDtypeStruct((num_tiles, 1, _TN), jnp.float32),
        grid=(num_tiles,),
        in_specs=[
            pl.BlockSpec((None, 1, _TN), lambda t: (t, 0, 0),
                         memory_space=pltpu.SMEM),
            pl.BlockSpec((None, 1, _TN), lambda t: (t, 0, 0),
                         memory_space=pltpu.SMEM),
            pl.BlockSpec((p_rows, 1, d), lambda t: (0, 0, 0)),
            pl.BlockSpec((u_rows, 1, d), lambda t: (0, 0, 0)),
        ],
        out_specs=pl.BlockSpec((None, 1, _TN), lambda t: (t, 0, 0)),
        scratch_shapes=[pltpu.VMEM((_TN, d), jnp.float32)],
        compiler_params=pltpu.CompilerParams(
            dimension_semantics=("parallel",),
            vmem_limit_bytes=vmem_limit,
        ),
        cost_estimate=cost,
    )(prod_ids, user_ids, ptab, utab)
    return out.reshape(n_pad)[:n]


# speedup vs baseline: 1.3026x; 1.0768x over previous
import jax
import jax.numpy as jnp
from jax.experimental import pallas as pl
from jax.experimental.pallas import tpu as pltpu

_TN = 2048
_CHUNK = 128


def _round_up(x, m):
    return ((x + m - 1) // m) * m


def _gather_dot_kernel(pids_ref, uids_ref, ptab_ref, utab_ref, out_ref, c_ref):
    tn = out_ref.shape[1]

    def chunk_body(c, carry):
        base = c * _CHUNK
        for j in range(_CHUNK // 8):
            rows = []
            for i in range(8):
                p = pids_ref[0, base + j * 8 + i]
                u = uids_ref[0, base + j * 8 + i]
                rows.append(ptab_ref[p] * utab_ref[u])
            blk = jnp.concatenate(rows, axis=0)
            c_ref[pl.ds(pl.multiple_of(base + j * 8, 8), 8), :] = blk
        return carry

    half = tn // (2 * _CHUNK)
    ones = jnp.ones((1, 128), jnp.float32)
    dims = (((1,), (1,)), ((), ()))

    jax.lax.fori_loop(0, half, chunk_body, 0)
    out_ref[0, pl.ds(0, tn // 2)] = jax.lax.dot_general(
        ones, c_ref[pl.ds(0, tn // 2), :], dims,
        preferred_element_type=jnp.float32)[0]
    jax.lax.fori_loop(half, 2 * half, chunk_body, 0)
    out_ref[0, pl.ds(tn // 2, tn // 2)] = jax.lax.dot_general(
        ones, c_ref[pl.ds(tn // 2, tn // 2), :], dims,
        preferred_element_type=jnp.float32)[0]


def kernel(products, users, product_embedding, user_embedding):
    n = products.shape[0]
    p_rows, d = product_embedding.shape
    u_rows, d_u = user_embedding.shape
    assert d == d_u == 128

    n_pad = _round_up(n, _TN)
    num_tiles = n_pad // _TN

    def prep_ids(ids, rows):
        ids = jnp.clip(jnp.asarray(ids).astype(jnp.int32), 0, rows - 1)
        ids = jnp.pad(ids, (0, n_pad - n))
        return ids.reshape(num_tiles, 1, _TN)

    prod_ids = prep_ids(products, p_rows)
    user_ids = prep_ids(users, u_rows)

    ptab = product_embedding.astype(jnp.float32).reshape(p_rows, 1, d)
    utab = user_embedding.astype(jnp.float32).reshape(u_rows, 1, d)

    table_bytes = (p_rows + u_rows) * d * 4
    vmem_limit = min(int(2 * table_bytes + 4 * _TN * 128 * 4 + (8 << 20)),
                     60 << 20)

    cost = pl.CostEstimate(
        flops=2 * n_pad * d + 2 * n_pad * d,
        transcendentals=0,
        bytes_accessed=2 * n_pad * 4 + 2 * table_bytes + n_pad * 4,
    )

    out = pl.pallas_call(
        _gather_dot_kernel,
        out_shape=jax.ShapeDtypeStruct((num_tiles, 1, _TN), jnp.float32),
        grid=(num_tiles,),
        in_specs=[
            pl.BlockSpec((None, 1, _TN), lambda t: (t, 0, 0),
                         memory_space=pltpu.SMEM),
            pl.BlockSpec((None, 1, _TN), lambda t: (t, 0, 0),
                         memory_space=pltpu.SMEM),
            pl.BlockSpec((p_rows, 1, d), lambda t: (0, 0, 0)),
            pl.BlockSpec((u_rows, 1, d), lambda t: (0, 0, 0)),
        ],
        out_specs=pl.BlockSpec((None, 1, _TN), lambda t: (t, 0, 0)),
        scratch_shapes=[pltpu.VMEM((_TN, d), jnp.float32)],
        compiler_params=pltpu.CompilerParams(
            dimension_semantics=("parallel",),
            vmem_limit_bytes=vmem_limit,
        ),
        cost_estimate=cost,
    )(prod_ids, user_ids, ptab, utab)
    return out.reshape(n_pad)[:n]


# speedup vs baseline: 1.3542x; 1.0396x over previous
import jax
import jax.numpy as jnp
from jax.experimental import pallas as pl
from jax.experimental.pallas import tpu as pltpu

_TN = 4096
_CHUNK = 128


def _round_up(x, m):
    return ((x + m - 1) // m) * m


def _gather_dot_kernel(pids_ref, uids_ref, ptab_ref, utab_ref, out_ref, c_ref):
    tn = out_ref.shape[1]

    def chunk_body(c, carry):
        base = c * _CHUNK
        for j in range(_CHUNK // 8):
            rows = []
            for i in range(8):
                p = pids_ref[0, base + j * 8 + i]
                u = uids_ref[0, base + j * 8 + i]
                rows.append(ptab_ref[p] * utab_ref[u])
            blk = jnp.concatenate(rows, axis=0)
            c_ref[pl.ds(pl.multiple_of(base + j * 8, 8), 8), :] = blk
        return carry

    half = tn // (2 * _CHUNK)
    ones = jnp.ones((1, 128), jnp.float32)
    dims = (((1,), (1,)), ((), ()))

    jax.lax.fori_loop(0, half, chunk_body, 0)
    out_ref[0, pl.ds(0, tn // 2)] = jax.lax.dot_general(
        ones, c_ref[pl.ds(0, tn // 2), :], dims,
        preferred_element_type=jnp.float32)[0]
    jax.lax.fori_loop(half, 2 * half, chunk_body, 0)
    out_ref[0, pl.ds(tn // 2, tn // 2)] = jax.lax.dot_general(
        ones, c_ref[pl.ds(tn // 2, tn // 2), :], dims,
        preferred_element_type=jnp.float32)[0]


def kernel(products, users, product_embedding, user_embedding):
    n = products.shape[0]
    p_rows, d = product_embedding.shape
    u_rows, d_u = user_embedding.shape
    assert d == d_u == 128

    n_pad = _round_up(n, _TN)
    num_tiles = n_pad // _TN

    def prep_ids(ids, rows):
        ids = jnp.clip(jnp.asarray(ids).astype(jnp.int32), 0, rows - 1)
        ids = jnp.pad(ids, (0, n_pad - n))
        return ids.reshape(num_tiles, 1, _TN)

    prod_ids = prep_ids(products, p_rows)
    user_ids = prep_ids(users, u_rows)

    ptab = product_embedding.astype(jnp.float32).reshape(p_rows, 1, d)
    utab = user_embedding.astype(jnp.float32).reshape(u_rows, 1, d)

    table_bytes = (p_rows + u_rows) * d * 4
    vmem_limit = min(int(2 * table_bytes + 4 * _TN * 128 * 4 + (8 << 20)),
                     60 << 20)

    cost = pl.CostEstimate(
        flops=2 * n_pad * d + 2 * n_pad * d,
        transcendentals=0,
        bytes_accessed=2 * n_pad * 4 + 2 * table_bytes + n_pad * 4,
    )

    out = pl.pallas_call(
        _gather_dot_kernel,
        out_shape=jax.ShapeDtypeStruct((num_tiles, 1, _TN), jnp.float32),
        grid=(num_tiles,),
        in_specs=[
            pl.BlockSpec((None, 1, _TN), lambda t: (t, 0, 0),
                         memory_space=pltpu.SMEM),
            pl.BlockSpec((None, 1, _TN), lambda t: (t, 0, 0),
                         memory_space=pltpu.SMEM),
            pl.BlockSpec((p_rows, 1, d), lambda t: (0, 0, 0)),
            pl.BlockSpec((u_rows, 1, d), lambda t: (0, 0, 0)),
        ],
        out_specs=pl.BlockSpec((None, 1, _TN), lambda t: (t, 0, 0)),
        scratch_shapes=[pltpu.VMEM((_TN, d), jnp.float32)],
        compiler_params=pltpu.CompilerParams(
            dimension_semantics=("parallel",),
            vmem_limit_bytes=vmem_limit,
        ),
        cost_estimate=cost,
    )(prod_ids, user_ids, ptab, utab)
    return out.reshape(n_pad)[:n]


# speedup vs baseline: 1.3809x; 1.0197x over previous
import jax
import jax.numpy as jnp
from jax.experimental import pallas as pl
from jax.experimental.pallas import tpu as pltpu

_TN = 8192
_CHUNK = 128


def _round_up(x, m):
    return ((x + m - 1) // m) * m


def _gather_dot_kernel(pids_ref, uids_ref, ptab_ref, utab_ref, out_ref, c_ref):
    tn = out_ref.shape[1]

    def chunk_body(c, carry):
        base = c * _CHUNK
        for j in range(_CHUNK // 8):
            rows = []
            for i in range(8):
                p = pids_ref[0, base + j * 8 + i]
                u = uids_ref[0, base + j * 8 + i]
                rows.append(ptab_ref[p] * utab_ref[u])
            blk = jnp.concatenate(rows, axis=0)
            c_ref[pl.ds(pl.multiple_of(base + j * 8, 8), 8), :] = blk
        return carry

    half = tn // (2 * _CHUNK)
    ones = jnp.ones((1, 128), jnp.float32)
    dims = (((1,), (1,)), ((), ()))

    jax.lax.fori_loop(0, half, chunk_body, 0)
    out_ref[0, pl.ds(0, tn // 2)] = jax.lax.dot_general(
        ones, c_ref[pl.ds(0, tn // 2), :], dims,
        preferred_element_type=jnp.float32)[0]
    jax.lax.fori_loop(half, 2 * half, chunk_body, 0)
    out_ref[0, pl.ds(tn // 2, tn // 2)] = jax.lax.dot_general(
        ones, c_ref[pl.ds(tn // 2, tn // 2), :], dims,
        preferred_element_type=jnp.float32)[0]


def kernel(products, users, product_embedding, user_embedding):
    n = products.shape[0]
    p_rows, d = product_embedding.shape
    u_rows, d_u = user_embedding.shape
    assert d == d_u == 128

    n_pad = _round_up(n, _TN)
    num_tiles = n_pad // _TN

    def prep_ids(ids, rows):
        ids = jnp.clip(jnp.asarray(ids).astype(jnp.int32), 0, rows - 1)
        ids = jnp.pad(ids, (0, n_pad - n))
        return ids.reshape(num_tiles, 1, _TN)

    prod_ids = prep_ids(products, p_rows)
    user_ids = prep_ids(users, u_rows)

    ptab = product_embedding.astype(jnp.float32).reshape(p_rows, 1, d)
    utab = user_embedding.astype(jnp.float32).reshape(u_rows, 1, d)

    table_bytes = (p_rows + u_rows) * d * 4
    vmem_limit = min(int(2 * table_bytes + 4 * _TN * 128 * 4 + (8 << 20)),
                     60 << 20)

    cost = pl.CostEstimate(
        flops=2 * n_pad * d + 2 * n_pad * d,
        transcendentals=0,
        bytes_accessed=2 * n_pad * 4 + 2 * table_bytes + n_pad * 4,
    )

    out = pl.pallas_call(
        _gather_dot_kernel,
        out_shape=jax.ShapeDtypeStruct((num_tiles, 1, _TN), jnp.float32),
        grid=(num_tiles,),
        in_specs=[
            pl.BlockSpec((None, 1, _TN), lambda t: (t, 0, 0),
                         memory_space=pltpu.SMEM),
            pl.BlockSpec((None, 1, _TN), lambda t: (t, 0, 0),
                         memory_space=pltpu.SMEM),
            pl.BlockSpec((p_rows, 1, d), lambda t: (0, 0, 0)),
            pl.BlockSpec((u_rows, 1, d), lambda t: (0, 0, 0)),
        ],
        out_specs=pl.BlockSpec((None, 1, _TN), lambda t: (t, 0, 0)),
        scratch_shapes=[pltpu.VMEM((_TN, d), jnp.float32)],
        compiler_params=pltpu.CompilerParams(
            dimension_semantics=("parallel",),
            vmem_limit_bytes=vmem_limit,
        ),
        cost_estimate=cost,
    )(prod_ids, user_ids, ptab, utab)
    return out.reshape(n_pad)[:n]


# speedup vs baseline: 1.3892x; 1.0060x over previous
import jax
import jax.numpy as jnp
from jax.experimental import pallas as pl
from jax.experimental.pallas import tpu as pltpu

_TN = 16384
_CHUNK = 128


def _round_up(x, m):
    return ((x + m - 1) // m) * m


def _gather_dot_kernel(pids_ref, uids_ref, ptab_ref, utab_ref, out_ref, c_ref):
    tn = out_ref.shape[1]

    def chunk_body(c, carry):
        base = c * _CHUNK
        for j in range(_CHUNK // 8):
            rows = []
            for i in range(8):
                p = pids_ref[0, base + j * 8 + i]
                u = uids_ref[0, base + j * 8 + i]
                rows.append(ptab_ref[p] * utab_ref[u])
            blk = jnp.concatenate(rows, axis=0)
            c_ref[pl.ds(pl.multiple_of(base + j * 8, 8), 8), :] = blk
        return carry

    half = tn // (2 * _CHUNK)
    ones = jnp.ones((1, 128), jnp.float32)
    dims = (((1,), (1,)), ((), ()))

    jax.lax.fori_loop(0, half, chunk_body, 0)
    out_ref[0, pl.ds(0, tn // 2)] = jax.lax.dot_general(
        ones, c_ref[pl.ds(0, tn // 2), :], dims,
        preferred_element_type=jnp.float32)[0]
    jax.lax.fori_loop(half, 2 * half, chunk_body, 0)
    out_ref[0, pl.ds(tn // 2, tn // 2)] = jax.lax.dot_general(
        ones, c_ref[pl.ds(tn // 2, tn // 2), :], dims,
        preferred_element_type=jnp.float32)[0]


def kernel(products, users, product_embedding, user_embedding):
    n = products.shape[0]
    p_rows, d = product_embedding.shape
    u_rows, d_u = user_embedding.shape
    assert d == d_u == 128

    n_pad = _round_up(n, _TN)
    num_tiles = n_pad // _TN

    def prep_ids(ids, rows):
        ids = jnp.clip(jnp.asarray(ids).astype(jnp.int32), 0, rows - 1)
        ids = jnp.pad(ids, (0, n_pad - n))
        return ids.reshape(num_tiles, 1, _TN)

    prod_ids = prep_ids(products, p_rows)
    user_ids = prep_ids(users, u_rows)

    ptab = product_embedding.astype(jnp.float32).reshape(p_rows, 1, d)
    utab = user_embedding.astype(jnp.float32).reshape(u_rows, 1, d)

    table_bytes = (p_rows + u_rows) * d * 4
    vmem_limit = min(int(2 * table_bytes + 4 * _TN * 128 * 4 + (8 << 20)),
                     60 << 20)

    cost = pl.CostEstimate(
        flops=2 * n_pad * d + 2 * n_pad * d,
        transcendentals=0,
        bytes_accessed=2 * n_pad * 4 + 2 * table_bytes + n_pad * 4,
    )

    out = pl.pallas_call(
        _gather_dot_kernel,
        out_shape=jax.ShapeDtypeStruct((num_tiles, 1, _TN), jnp.float32),
        grid=(num_tiles,),
        in_specs=[
            pl.BlockSpec((None, 1, _TN), lambda t: (t, 0, 0),
                         memory_space=pltpu.SMEM),
            pl.BlockSpec((None, 1, _TN), lambda t: (t, 0, 0),
                         memory_space=pltpu.SMEM),
            pl.BlockSpec((p_rows, 1, d), lambda t: (0, 0, 0)),
            pl.BlockSpec((u_rows, 1, d), lambda t: (0, 0, 0)),
        ],
        out_specs=pl.BlockSpec((None, 1, _TN), lambda t: (t, 0, 0)),
        scratch_shapes=[pltpu.VMEM((_TN, d), jnp.float32)],
        compiler_params=pltpu.CompilerParams(
            dimension_semantics=("parallel",),
            vmem_limit_bytes=vmem_limit,
        ),
        cost_estimate=cost,
    )(prod_ids, user_ids, ptab, utab)
    return out.reshape(n_pad)[:n]
